# baseline (speedup 1.0000x reference)
import functools

import jax
import jax.numpy as jnp
from jax import lax
from jax.experimental import pallas as pl
from jax.experimental.pallas import tpu as pltpu
from jax.experimental.pallas import tpu_sc as plsc

N = 10000
E = 320000
IN_CH = 128
H1 = 256
H2 = 128
NC, NS = 2, 16
NW = NC * NS
CH = 80
EPT = 10240
E_PAD = EPT * NW
NCH = EPT // CH
ACC_N = 10240
RPT = ACC_N // NS
DCH = 80
DNCH = E // NW // DCH
DEGW = 128
SLOPE = (1.0 / 8 + 1.0 / 3) / 2.0

_MESH = plsc.VectorSubcoreMesh(
    core_axis_name="c", subcore_axis_name="s", num_cores=NC, num_subcores=NS
)


@functools.partial(
    pl.kernel,
    out_type=jax.ShapeDtypeStruct((NC, ACC_N, DEGW), jnp.float32),
    mesh=_MESH,
    scratch_types=[
        pltpu.VMEM((DCH,), jnp.int32),
        pltpu.VMEM((DCH, DEGW), jnp.float32),
        pltpu.VMEM_SHARED((ACC_N, DEGW), jnp.float32),
    ],
)
def _deg_kernel(dst_hbm, ones_hbm, zero_hbm, out_hbm, didx, ones_v, acc):
    c = lax.axis_index("c")
    s = lax.axis_index("s")
    base0 = (c * NS + s) * (E // NW)
    r0 = s * RPT
    pltpu.sync_copy(ones_hbm, ones_v)
    pltpu.sync_copy(zero_hbm.at[pl.ds(r0, RPT)], acc.at[pl.ds(r0, RPT)])
    plsc.subcore_barrier()

    def body(i, carry):
        b = base0 + i * DCH
        pltpu.sync_copy(dst_hbm.at[pl.ds(b, DCH)], didx)
        pltpu.sync_copy(ones_v, acc.at[didx], add=True)
        return carry

    lax.fori_loop(0, DNCH, body, 0)
    plsc.subcore_barrier()
    pltpu.sync_copy(acc.at[pl.ds(r0, RPT)], out_hbm.at[c, pl.ds(r0, RPT)])


@functools.partial(
    pl.kernel,
    out_type=jax.ShapeDtypeStruct((NC, ACC_N, H2), jnp.float32),
    mesh=_MESH,
    scratch_types=[
        pltpu.VMEM((CH,), jnp.int32),
        pltpu.VMEM((CH,), jnp.int32),
        pltpu.VMEM((CH,), jnp.int32),
        pltpu.VMEM((CH,), jnp.int32),
        pltpu.VMEM((CH, H2), jnp.float32),
        pltpu.VMEM((CH, H2), jnp.float32),
        pltpu.VMEM_SHARED((ACC_N, H2), jnp.float32),
        pltpu.SemaphoreType.DMA,
        pltpu.SemaphoreType.DMA,
    ],
)
def _spmm_kernel(table_hbm, src_hbm, dst_hbm, zero_hbm, out_hbm,
                 sidx0, sidx1, didx0, didx1, rows0, rows1, acc,
                 gsem0, gsem1):
    c = lax.axis_index("c")
    s = lax.axis_index("s")
    base0 = (c * NS + s) * EPT
    r0 = s * RPT
    pltpu.sync_copy(zero_hbm.at[pl.ds(r0, RPT)], acc.at[pl.ds(r0, RPT)])
    plsc.subcore_barrier()

    def load_idx(b, si, di):
        pltpu.sync_copy(src_hbm.at[pl.ds(b, CH)], si)
        pltpu.sync_copy(dst_hbm.at[pl.ds(b, CH)], di)

    load_idx(base0, sidx0, didx0)
    pltpu.async_copy(table_hbm.at[sidx0], rows0, gsem0)

    def body(j, carry):
        b = base0 + 2 * j * CH
        pltpu.make_async_copy(table_hbm.at[sidx0], rows0, gsem0).wait()
        load_idx(b + CH, sidx1, didx1)
        pltpu.async_copy(table_hbm.at[sidx1], rows1, gsem1)
        pltpu.sync_copy(rows0, acc.at[didx0], add=True)
        pltpu.make_async_copy(table_hbm.at[sidx1], rows1, gsem1).wait()
        load_idx(b + 2 * CH, sidx0, didx0)
        pltpu.async_copy(table_hbm.at[sidx0], rows0, gsem0)
        pltpu.sync_copy(rows1, acc.at[didx1], add=True)
        return carry

    lax.fori_loop(0, NCH // 2 - 1, body, 0)
    b = base0 + (NCH - 2) * CH
    pltpu.make_async_copy(table_hbm.at[sidx0], rows0, gsem0).wait()
    load_idx(b + CH, sidx1, didx1)
    pltpu.async_copy(table_hbm.at[sidx1], rows1, gsem1)
    pltpu.sync_copy(rows0, acc.at[didx0], add=True)
    pltpu.make_async_copy(table_hbm.at[sidx1], rows1, gsem1).wait()
    pltpu.sync_copy(rows1, acc.at[didx1], add=True)
    plsc.subcore_barrier()
    pltpu.sync_copy(acc.at[pl.ds(r0, RPT)], out_hbm.at[c, pl.ds(r0, RPT)])


_RB = 2000


def _rsqrt_deg(degp_ref):
    deg = degp_ref[0, :, 0:1] + degp_ref[1, :, 0:1] + 1.0
    return lax.rsqrt(deg)


def _tc1_body(x_ref, w1_ref, degp_ref, h1a_ref, h1b_ref):
    xh = jnp.dot(x_ref[...], w1_ref[...], preferred_element_type=jnp.float32)
    d = _rsqrt_deg(degp_ref)
    h = xh * d
    h1a_ref[...] = h[:, :H2]
    h1b_ref[...] = h[:, H2:]


def _tc2_body(a1a_ref, a1b_ref, h1a_ref, h1b_ref, degp_ref,
              w2a_ref, w2b_ref, b1_ref, h2_ref):
    d = _rsqrt_deg(degp_ref)
    ua = d * (a1a_ref[0] + a1a_ref[1] + h1a_ref[...]) + b1_ref[:, :H2]
    ub = d * (a1b_ref[0] + a1b_ref[1] + h1b_ref[...]) + b1_ref[:, H2:]
    ra = jnp.where(ua >= 0, ua, ua * SLOPE)
    rb = jnp.where(ub >= 0, ub, ub * SLOPE)
    xh2 = (jnp.dot(ra, w2a_ref[...], preferred_element_type=jnp.float32)
           + jnp.dot(rb, w2b_ref[...], preferred_element_type=jnp.float32))
    h2_ref[...] = xh2 * d


def _tc3_body(a2_ref, h2_ref, degp_ref, b2_ref, z_ref):
    d = _rsqrt_deg(degp_ref)
    z_ref[...] = d * (a2_ref[0] + a2_ref[1] + h2_ref[...]) + b2_ref[...]


def _row_spec(w):
    return pl.BlockSpec((_RB, w), lambda i: (i, 0))


def _part_spec(w):
    return pl.BlockSpec((NC, _RB, w), lambda i: (0, i, 0))


_DEG_SPEC = pl.BlockSpec((NC, _RB, DEGW), lambda i: (0, i, 0))
_GRID = (N // _RB,)

_tc1 = pl.pallas_call(
    _tc1_body,
    grid=_GRID,
    in_specs=[
        _row_spec(IN_CH),
        pl.BlockSpec((IN_CH, H1), lambda i: (0, 0)),
        _DEG_SPEC,
    ],
    out_specs=[_row_spec(H2), _row_spec(H2)],
    out_shape=[
        jax.ShapeDtypeStruct((N, H2), jnp.float32),
        jax.ShapeDtypeStruct((N, H2), jnp.float32),
    ],
)

_tc2 = pl.pallas_call(
    _tc2_body,
    grid=_GRID,
    in_specs=[
        _part_spec(H2),
        _part_spec(H2),
        _row_spec(H2),
        _row_spec(H2),
        _DEG_SPEC,
        pl.BlockSpec((H2, H2), lambda i: (0, 0)),
        pl.BlockSpec((H2, H2), lambda i: (0, 0)),
        pl.BlockSpec((1, H1), lambda i: (0, 0)),
    ],
    out_specs=_row_spec(H2),
    out_shape=jax.ShapeDtypeStruct((N, H2), jnp.float32),
)

_tc3 = pl.pallas_call(
    _tc3_body,
    grid=_GRID,
    in_specs=[
        _part_spec(H2),
        _row_spec(H2),
        _DEG_SPEC,
        pl.BlockSpec((1, H2), lambda i: (0, 0)),
    ],
    out_specs=_row_spec(H2),
    out_shape=jax.ShapeDtypeStruct((N, H2), jnp.float32),
)


def kernel(x, edge_index, W1, b1, W2, b2):
    src = edge_index[0].astype(jnp.int32)
    dst = edge_index[1].astype(jnp.int32)
    pad = E_PAD - E
    src_p = jnp.concatenate([src, jnp.zeros((pad,), jnp.int32)])
    dst_p = jnp.concatenate(
        [dst, N + (jnp.arange(pad, dtype=jnp.int32) % (ACC_N - N))])
    ones_rows = jnp.ones((DCH, DEGW), jnp.float32)
    zdeg = jnp.zeros((ACC_N, DEGW), jnp.float32)
    zacc = jnp.zeros((ACC_N, H2), jnp.float32)

    degp = _deg_kernel(dst, ones_rows, zdeg)
    h1a, h1b = _tc1(x, W1, degp)
    agg1a = _spmm_kernel(h1a, src_p, dst_p, zacc)
    agg1b = _spmm_kernel(h1b, src_p, dst_p, zacc)
    h2 = _tc2(agg1a, agg1b, h1a, h1b, degp,
              W2[:H2], W2[H2:], b1.reshape(1, H1))
    agg2 = _spmm_kernel(h2, src_p, dst_p, zacc)
    z = _tc3(agg2, h2, degp, b2.reshape(1, H2))
    return z


# speedup vs baseline: 2.1138x; 2.1138x over previous
import functools

import jax
import jax.numpy as jnp
from jax import lax
from jax.experimental import pallas as pl
from jax.experimental.pallas import tpu as pltpu
from jax.experimental.pallas import tpu_sc as plsc

N = 10000
E = 320000
IN_CH = 128
H1 = 256
H2 = 128
NC, NS = 2, 16
NW = NC * NS
CH = 80
EPT = E // NW
NCH = EPT // CH
ACC_N = 10240
RPT = ACC_N // NS
DCH = 80
DNCH = E // NW // DCH
DEGW = 128
SLOPE = (1.0 / 8 + 1.0 / 3) / 2.0

_MESH = plsc.VectorSubcoreMesh(
    core_axis_name="c", subcore_axis_name="s", num_cores=NC, num_subcores=NS
)


@functools.partial(
    pl.kernel,
    out_type=jax.ShapeDtypeStruct((NC, ACC_N, DEGW), jnp.float32),
    mesh=_MESH,
    scratch_types=[
        pltpu.VMEM((DCH,), jnp.int32),
        pltpu.VMEM((DCH, DEGW), jnp.float32),
        pltpu.VMEM_SHARED((ACC_N, DEGW), jnp.float32),
    ],
)
def _deg_kernel(dst_hbm, ones_hbm, zero_hbm, out_hbm, didx, ones_v, acc):
    c = lax.axis_index("c")
    s = lax.axis_index("s")
    base0 = (c * NS + s) * (E // NW)
    r0 = s * RPT
    pltpu.sync_copy(ones_hbm, ones_v)
    pltpu.sync_copy(zero_hbm.at[pl.ds(r0, RPT)], acc.at[pl.ds(r0, RPT)])
    plsc.subcore_barrier()

    def body(i, carry):
        b = base0 + i * DCH
        pltpu.sync_copy(dst_hbm.at[pl.ds(b, DCH)], didx)
        pltpu.sync_copy(ones_v, acc.at[didx], add=True)
        return carry

    lax.fori_loop(0, DNCH, body, 0)
    plsc.subcore_barrier()
    pltpu.sync_copy(acc.at[pl.ds(r0, RPT)], out_hbm.at[c, pl.ds(r0, RPT)])


@functools.partial(
    pl.kernel,
    out_type=jax.ShapeDtypeStruct((NC, ACC_N, H2), jnp.float32),
    mesh=_MESH,
    scratch_types=[
        pltpu.VMEM((CH,), jnp.int32),
        pltpu.VMEM((CH,), jnp.int32),
        pltpu.VMEM((CH,), jnp.int32),
        pltpu.VMEM((CH,), jnp.int32),
        pltpu.VMEM((CH, H2), jnp.float32),
        pltpu.VMEM((CH, H2), jnp.float32),
        pltpu.VMEM_SHARED((ACC_N, H2), jnp.float32),
        pltpu.SemaphoreType.DMA,
        pltpu.SemaphoreType.DMA,
    ],
)
def _spmm_kernel(table_hbm, src_hbm, dst_hbm, zero_hbm, out_hbm,
                 sidx0, sidx1, didx0, didx1, rows0, rows1, acc,
                 gsem0, gsem1):
    c = lax.axis_index("c")
    s = lax.axis_index("s")
    base0 = (c * NS + s) * EPT
    r0 = s * RPT
    pltpu.sync_copy(zero_hbm.at[pl.ds(r0, RPT)], acc.at[pl.ds(r0, RPT)])
    plsc.subcore_barrier()

    def load_idx(b, si, di):
        pltpu.sync_copy(src_hbm.at[pl.ds(b, CH)], si)
        pltpu.sync_copy(dst_hbm.at[pl.ds(b, CH)], di)

    load_idx(base0, sidx0, didx0)
    pltpu.async_copy(table_hbm.at[sidx0], rows0, gsem0)

    def body(j, carry):
        b = base0 + 2 * j * CH
        pltpu.make_async_copy(table_hbm.at[sidx0], rows0, gsem0).wait()
        load_idx(b + CH, sidx1, didx1)
        pltpu.async_copy(table_hbm.at[sidx1], rows1, gsem1)
        pltpu.sync_copy(rows0, acc.at[didx0], add=True)
        pltpu.make_async_copy(table_hbm.at[sidx1], rows1, gsem1).wait()
        load_idx(b + 2 * CH, sidx0, didx0)
        pltpu.async_copy(table_hbm.at[sidx0], rows0, gsem0)
        pltpu.sync_copy(rows1, acc.at[didx1], add=True)
        return carry

    lax.fori_loop(0, NCH // 2, body, 0)
    pltpu.make_async_copy(table_hbm.at[sidx0], rows0, gsem0).wait()
    pltpu.sync_copy(rows0, acc.at[didx0], add=True)
    plsc.subcore_barrier()
    pltpu.sync_copy(acc.at[pl.ds(r0, RPT)], out_hbm.at[c, pl.ds(r0, RPT)])


_RB = 2000


def _rsqrt_deg(degp_ref):
    deg = degp_ref[0, :, 0:1] + degp_ref[1, :, 0:1] + 1.0
    return lax.rsqrt(deg)


def _tc1_body(x_ref, w1_ref, degp_ref, h1a_ref, h1b_ref):
    xh = jnp.dot(x_ref[...], w1_ref[...], preferred_element_type=jnp.float32)
    d = _rsqrt_deg(degp_ref)
    h = xh * d
    h1a_ref[...] = h[:, :H2]
    h1b_ref[...] = h[:, H2:]


def _tc2_body(a1a_ref, a1b_ref, h1a_ref, h1b_ref, degp_ref,
              w2a_ref, w2b_ref, b1_ref, h2_ref):
    d = _rsqrt_deg(degp_ref)
    ua = d * (a1a_ref[0] + a1a_ref[1] + h1a_ref[...]) + b1_ref[:, :H2]
    ub = d * (a1b_ref[0] + a1b_ref[1] + h1b_ref[...]) + b1_ref[:, H2:]
    ra = jnp.where(ua >= 0, ua, ua * SLOPE)
    rb = jnp.where(ub >= 0, ub, ub * SLOPE)
    xh2 = (jnp.dot(ra, w2a_ref[...], preferred_element_type=jnp.float32)
           + jnp.dot(rb, w2b_ref[...], preferred_element_type=jnp.float32))
    h2_ref[...] = xh2 * d


def _tc3_body(a2_ref, h2_ref, degp_ref, b2_ref, z_ref):
    d = _rsqrt_deg(degp_ref)
    z_ref[...] = d * (a2_ref[0] + a2_ref[1] + h2_ref[...]) + b2_ref[...]


def _row_spec(w):
    return pl.BlockSpec((_RB, w), lambda i: (i, 0))


def _part_spec(w):
    return pl.BlockSpec((NC, _RB, w), lambda i: (0, i, 0))


_DEG_SPEC = pl.BlockSpec((NC, _RB, DEGW), lambda i: (0, i, 0))
_GRID = (N // _RB,)

_tc1 = pl.pallas_call(
    _tc1_body,
    grid=_GRID,
    in_specs=[
        _row_spec(IN_CH),
        pl.BlockSpec((IN_CH, H1), lambda i: (0, 0)),
        _DEG_SPEC,
    ],
    out_specs=[_row_spec(H2), _row_spec(H2)],
    out_shape=[
        jax.ShapeDtypeStruct((N, H2), jnp.float32),
        jax.ShapeDtypeStruct((N, H2), jnp.float32),
    ],
)

_tc2 = pl.pallas_call(
    _tc2_body,
    grid=_GRID,
    in_specs=[
        _part_spec(H2),
        _part_spec(H2),
        _row_spec(H2),
        _row_spec(H2),
        _DEG_SPEC,
        pl.BlockSpec((H2, H2), lambda i: (0, 0)),
        pl.BlockSpec((H2, H2), lambda i: (0, 0)),
        pl.BlockSpec((1, H1), lambda i: (0, 0)),
    ],
    out_specs=_row_spec(H2),
    out_shape=jax.ShapeDtypeStruct((N, H2), jnp.float32),
)

_tc3 = pl.pallas_call(
    _tc3_body,
    grid=_GRID,
    in_specs=[
        _part_spec(H2),
        _row_spec(H2),
        _DEG_SPEC,
        pl.BlockSpec((1, H2), lambda i: (0, 0)),
    ],
    out_specs=_row_spec(H2),
    out_shape=jax.ShapeDtypeStruct((N, H2), jnp.float32),
)


def kernel(x, edge_index, W1, b1, W2, b2):
    src = edge_index[0].astype(jnp.int32)
    dst = edge_index[1].astype(jnp.int32)
    ones_rows = jnp.ones((DCH, DEGW), jnp.float32)
    zdeg = jnp.zeros((ACC_N, DEGW), jnp.float32)
    zacc = jnp.zeros((ACC_N, H2), jnp.float32)

    degp = _deg_kernel(dst, ones_rows, zdeg)
    h1a, h1b = _tc1(x, W1, degp)
    agg1a = _spmm_kernel(h1a, src, dst, zacc)
    agg1b = _spmm_kernel(h1b, src, dst, zacc)
    h2 = _tc2(agg1a, agg1b, h1a, h1b, degp,
              W2[:H2], W2[H2:], b1.reshape(1, H1))
    agg2 = _spmm_kernel(h2, src, dst, zacc)
    z = _tc3(agg2, h2, degp, b2.reshape(1, H2))
    return z


# speedup vs baseline: 2.6186x; 1.2388x over previous
import functools

import jax
import jax.numpy as jnp
from jax import lax
from jax.experimental import pallas as pl
from jax.experimental.pallas import tpu as pltpu
from jax.experimental.pallas import tpu_sc as plsc

N = 10000
E = 320000
IN_CH = 128
H1 = 256
H2 = 128
NC, NS = 2, 16
NW = NC * NS
CH = 80
EPT = E // NW
NCH = EPT // CH
ACC_N = 10240
RPT = ACC_N // NS
DCH = 80
DNCH = E // NW // DCH
DEGW = 128
SLOPE = (1.0 / 8 + 1.0 / 3) / 2.0

_MESH = plsc.VectorSubcoreMesh(
    core_axis_name="c", subcore_axis_name="s", num_cores=NC, num_subcores=NS
)


@functools.partial(
    pl.kernel,
    out_type=jax.ShapeDtypeStruct((NC, ACC_N, DEGW), jnp.float32),
    mesh=_MESH,
    scratch_types=[
        pltpu.VMEM((DCH,), jnp.int32),
        pltpu.VMEM((DCH, DEGW), jnp.float32),
        pltpu.VMEM_SHARED((ACC_N, DEGW), jnp.float32),
    ],
)
def _deg_kernel(dst_hbm, ones_hbm, zero_hbm, out_hbm, didx, ones_v, acc):
    c = lax.axis_index("c")
    s = lax.axis_index("s")
    base0 = (c * NS + s) * (E // NW)
    r0 = s * RPT
    pltpu.sync_copy(ones_hbm, ones_v)
    pltpu.sync_copy(zero_hbm.at[pl.ds(r0, RPT)], acc.at[pl.ds(r0, RPT)])
    plsc.subcore_barrier()

    def body(i, carry):
        b = base0 + i * DCH
        pltpu.sync_copy(dst_hbm.at[pl.ds(b, DCH)], didx)
        pltpu.sync_copy(ones_v, acc.at[didx], add=True)
        return carry

    lax.fori_loop(0, DNCH, body, 0)
    plsc.subcore_barrier()
    pltpu.sync_copy(acc.at[pl.ds(r0, RPT)], out_hbm.at[c, pl.ds(r0, RPT)])


@functools.partial(
    pl.kernel,
    out_type=jax.ShapeDtypeStruct((NC, ACC_N, H2), jnp.float32),
    mesh=_MESH,
    scratch_types=[
        pltpu.VMEM((CH,), jnp.int32),
        pltpu.VMEM((CH,), jnp.int32),
        pltpu.VMEM((CH,), jnp.int32),
        pltpu.VMEM((CH,), jnp.int32),
        pltpu.VMEM((CH, H2), jnp.float32),
        pltpu.VMEM((CH, H2), jnp.float32),
        pltpu.VMEM_SHARED((ACC_N, H2), jnp.float32),
        pltpu.SemaphoreType.DMA,
        pltpu.SemaphoreType.DMA,
        pltpu.SemaphoreType.DMA,
        pltpu.SemaphoreType.DMA,
    ],
)
def _spmm_kernel(table_hbm, src_hbm, dst_hbm, zero_hbm, out_hbm,
                 sidx0, sidx1, didx0, didx1, rows0, rows1, acc,
                 gsem0, gsem1, ssem0, ssem1):
    c = lax.axis_index("c")
    s = lax.axis_index("s")
    base0 = (c * NS + s) * EPT
    r0 = s * RPT
    pltpu.sync_copy(zero_hbm.at[pl.ds(r0, RPT)], acc.at[pl.ds(r0, RPT)])
    plsc.subcore_barrier()

    def load_idx(b, si, di):
        pltpu.sync_copy(src_hbm.at[pl.ds(b, CH)], si)
        pltpu.sync_copy(dst_hbm.at[pl.ds(b, CH)], di)

    load_idx(base0, sidx0, didx0)
    pltpu.async_copy(table_hbm.at[sidx0], rows0, gsem0)
    load_idx(base0 + CH, sidx1, didx1)
    pltpu.async_copy(table_hbm.at[sidx1], rows1, gsem1)

    def body(j, carry):
        b2 = base0 + (2 * j + 2) * CH
        pltpu.make_async_copy(table_hbm.at[sidx0], rows0, gsem0).wait()
        pltpu.async_copy(rows0, acc.at[didx0], ssem0, add=True)
        pltpu.make_async_copy(table_hbm.at[sidx1], rows1, gsem1).wait()
        pltpu.async_copy(rows1, acc.at[didx1], ssem1, add=True)
        pltpu.make_async_copy(rows0, acc.at[didx0], ssem0).wait()
        load_idx(b2, sidx0, didx0)
        pltpu.async_copy(table_hbm.at[sidx0], rows0, gsem0)
        pltpu.make_async_copy(rows1, acc.at[didx1], ssem1).wait()
        load_idx(b2 + CH, sidx1, didx1)
        pltpu.async_copy(table_hbm.at[sidx1], rows1, gsem1)
        return carry

    lax.fori_loop(0, NCH // 2 - 1, body, 0)
    pltpu.make_async_copy(table_hbm.at[sidx0], rows0, gsem0).wait()
    pltpu.async_copy(rows0, acc.at[didx0], ssem0, add=True)
    pltpu.make_async_copy(table_hbm.at[sidx1], rows1, gsem1).wait()
    pltpu.async_copy(rows1, acc.at[didx1], ssem1, add=True)
    pltpu.make_async_copy(rows0, acc.at[didx0], ssem0).wait()
    load_idx(base0 + (NCH - 1) * CH, sidx0, didx0)
    pltpu.async_copy(table_hbm.at[sidx0], rows0, gsem0)
    pltpu.make_async_copy(rows1, acc.at[didx1], ssem1).wait()
    pltpu.make_async_copy(table_hbm.at[sidx0], rows0, gsem0).wait()
    pltpu.sync_copy(rows0, acc.at[didx0], add=True)
    plsc.subcore_barrier()
    pltpu.sync_copy(acc.at[pl.ds(r0, RPT)], out_hbm.at[c, pl.ds(r0, RPT)])


_RB = 2000


def _rsqrt_deg(degp_ref):
    deg = degp_ref[0, :, 0:1] + degp_ref[1, :, 0:1] + 1.0
    return lax.rsqrt(deg)


def _tc1_body(x_ref, w1_ref, degp_ref, h1a_ref, h1b_ref):
    xh = jnp.dot(x_ref[...], w1_ref[...], preferred_element_type=jnp.float32)
    d = _rsqrt_deg(degp_ref)
    h = xh * d
    h1a_ref[...] = h[:, :H2]
    h1b_ref[...] = h[:, H2:]


def _tc2_body(a1a_ref, a1b_ref, h1a_ref, h1b_ref, degp_ref,
              w2a_ref, w2b_ref, b1_ref, h2_ref):
    d = _rsqrt_deg(degp_ref)
    ua = d * (a1a_ref[0] + a1a_ref[1] + h1a_ref[...]) + b1_ref[:, :H2]
    ub = d * (a1b_ref[0] + a1b_ref[1] + h1b_ref[...]) + b1_ref[:, H2:]
    ra = jnp.where(ua >= 0, ua, ua * SLOPE)
    rb = jnp.where(ub >= 0, ub, ub * SLOPE)
    xh2 = (jnp.dot(ra, w2a_ref[...], preferred_element_type=jnp.float32)
           + jnp.dot(rb, w2b_ref[...], preferred_element_type=jnp.float32))
    h2_ref[...] = xh2 * d


def _tc3_body(a2_ref, h2_ref, degp_ref, b2_ref, z_ref):
    d = _rsqrt_deg(degp_ref)
    z_ref[...] = d * (a2_ref[0] + a2_ref[1] + h2_ref[...]) + b2_ref[...]


def _row_spec(w):
    return pl.BlockSpec((_RB, w), lambda i: (i, 0))


def _part_spec(w):
    return pl.BlockSpec((NC, _RB, w), lambda i: (0, i, 0))


_DEG_SPEC = pl.BlockSpec((NC, _RB, DEGW), lambda i: (0, i, 0))
_GRID = (N // _RB,)

_tc1 = pl.pallas_call(
    _tc1_body,
    grid=_GRID,
    in_specs=[
        _row_spec(IN_CH),
        pl.BlockSpec((IN_CH, H1), lambda i: (0, 0)),
        _DEG_SPEC,
    ],
    out_specs=[_row_spec(H2), _row_spec(H2)],
    out_shape=[
        jax.ShapeDtypeStruct((N, H2), jnp.float32),
        jax.ShapeDtypeStruct((N, H2), jnp.float32),
    ],
)

_tc2 = pl.pallas_call(
    _tc2_body,
    grid=_GRID,
    in_specs=[
        _part_spec(H2),
        _part_spec(H2),
        _row_spec(H2),
        _row_spec(H2),
        _DEG_SPEC,
        pl.BlockSpec((H2, H2), lambda i: (0, 0)),
        pl.BlockSpec((H2, H2), lambda i: (0, 0)),
        pl.BlockSpec((1, H1), lambda i: (0, 0)),
    ],
    out_specs=_row_spec(H2),
    out_shape=jax.ShapeDtypeStruct((N, H2), jnp.float32),
)

_tc3 = pl.pallas_call(
    _tc3_body,
    grid=_GRID,
    in_specs=[
        _part_spec(H2),
        _row_spec(H2),
        _DEG_SPEC,
        pl.BlockSpec((1, H2), lambda i: (0, 0)),
    ],
    out_specs=_row_spec(H2),
    out_shape=jax.ShapeDtypeStruct((N, H2), jnp.float32),
)


def kernel(x, edge_index, W1, b1, W2, b2):
    src = edge_index[0].astype(jnp.int32)
    dst = edge_index[1].astype(jnp.int32)
    ones_rows = jnp.ones((DCH, DEGW), jnp.float32)
    zdeg = jnp.zeros((ACC_N, DEGW), jnp.float32)
    zacc = jnp.zeros((ACC_N, H2), jnp.float32)

    degp = _deg_kernel(dst, ones_rows, zdeg)
    h1a, h1b = _tc1(x, W1, degp)
    agg1a = _spmm_kernel(h1a, src, dst, zacc)
    agg1b = _spmm_kernel(h1b, src, dst, zacc)
    h2 = _tc2(agg1a, agg1b, h1a, h1b, degp,
              W2[:H2], W2[H2:], b1.reshape(1, H1))
    agg2 = _spmm_kernel(h2, src, dst, zacc)
    z = _tc3(agg2, h2, degp, b2.reshape(1, H2))
    return z


# speedup vs baseline: 3.3713x; 1.2875x over previous
import functools

import jax
import jax.numpy as jnp
from jax import lax
from jax.experimental import pallas as pl
from jax.experimental.pallas import tpu as pltpu
from jax.experimental.pallas import tpu_sc as plsc

N = 10000
E = 320000
IN_CH = 128
H1 = 256
H2 = 128
NC, NS = 2, 16
NW = NC * NS
CH = 80
EPT = E // NW
NCH = EPT // CH
ACC_N = 10240
RPT = ACC_N // NS
DCH = 80
DNCH = E // NW // DCH
DEGW = 128
SLOPE = (1.0 / 8 + 1.0 / 3) / 2.0

_MESH = plsc.VectorSubcoreMesh(
    core_axis_name="c", subcore_axis_name="s", num_cores=NC, num_subcores=NS
)


@functools.partial(
    pl.kernel,
    out_type=jax.ShapeDtypeStruct((NC, ACC_N, DEGW), jnp.float32),
    mesh=_MESH,
    scratch_types=[
        pltpu.VMEM((NCH, CH), jnp.int32),
        pltpu.VMEM((DCH, DEGW), jnp.float32),
        pltpu.VMEM_SHARED((ACC_N, DEGW), jnp.float32),
        pltpu.SemaphoreType.DMA,
        pltpu.SemaphoreType.DMA,
    ],
)
def _deg_kernel(dst3_hbm, ones_hbm, zero_hbm, out_hbm, didx_all, ones_v, acc,
                ssem0, ssem1):
    c = lax.axis_index("c")
    s = lax.axis_index("s")
    w = c * NS + s
    r0 = s * RPT
    pltpu.sync_copy(ones_hbm, ones_v)
    pltpu.sync_copy(dst3_hbm.at[w], didx_all)
    pltpu.sync_copy(zero_hbm.at[pl.ds(r0, RPT)], acc.at[pl.ds(r0, RPT)])
    plsc.subcore_barrier()

    pltpu.async_copy(ones_v, acc.at[didx_all.at[0]], ssem0, add=True)
    pltpu.async_copy(ones_v, acc.at[didx_all.at[1]], ssem1, add=True)

    def body(j, carry):
        pltpu.make_async_copy(ones_v, acc.at[didx_all.at[2 * j]], ssem0).wait()
        pltpu.async_copy(ones_v, acc.at[didx_all.at[2 * j + 2]], ssem0, add=True)
        pltpu.make_async_copy(ones_v, acc.at[didx_all.at[2 * j + 1]], ssem1).wait()
        pltpu.async_copy(ones_v, acc.at[didx_all.at[2 * j + 3]], ssem1, add=True)
        return carry

    lax.fori_loop(0, NCH // 2 - 1, body, 0)
    pltpu.make_async_copy(ones_v, acc.at[didx_all.at[NCH - 3]], ssem0).wait()
    pltpu.async_copy(ones_v, acc.at[didx_all.at[NCH - 1]], ssem0, add=True)
    pltpu.make_async_copy(ones_v, acc.at[didx_all.at[NCH - 2]], ssem1).wait()
    pltpu.make_async_copy(ones_v, acc.at[didx_all.at[NCH - 1]], ssem0).wait()
    plsc.subcore_barrier()
    pltpu.sync_copy(acc.at[pl.ds(r0, RPT)], out_hbm.at[c, pl.ds(r0, RPT)])


@functools.partial(
    pl.kernel,
    out_type=jax.ShapeDtypeStruct((NC, ACC_N, H2), jnp.float32),
    mesh=_MESH,
    scratch_types=[
        pltpu.VMEM((2, CH), jnp.int32),
        pltpu.VMEM((2, CH), jnp.int32),
        pltpu.VMEM((2, CH), jnp.int32),
        pltpu.VMEM((2, CH), jnp.int32),
        pltpu.VMEM((CH, H2), jnp.float32),
        pltpu.VMEM((CH, H2), jnp.float32),
        pltpu.VMEM_SHARED((ACC_N, H2), jnp.float32),
        pltpu.SemaphoreType.DMA,
        pltpu.SemaphoreType.DMA,
        pltpu.SemaphoreType.DMA,
        pltpu.SemaphoreType.DMA,
        pltpu.SemaphoreType.DMA,
        pltpu.SemaphoreType.DMA,
        pltpu.SemaphoreType.DMA,
        pltpu.SemaphoreType.DMA,
    ],
)
def _spmm_kernel(table_hbm, idx2_hbm, zero_hbm, out_hbm,
                 ib0, ib1, ib2, ib3, rows0, rows1, acc,
                 gsem0, gsem1, ssem0, ssem1, isem0, isem1, isem2, isem3):
    c = lax.axis_index("c")
    s = lax.axis_index("s")
    w = c * NS + s
    r0 = s * RPT
    pltpu.sync_copy(zero_hbm.at[pl.ds(r0, RPT)], acc.at[pl.ds(r0, RPT)])
    plsc.subcore_barrier()

    def idx_load(i, ib, isem):
        pltpu.async_copy(idx2_hbm.at[w, i], ib, isem)

    def idx_wait(i, ib, isem):
        pltpu.make_async_copy(idx2_hbm.at[w, i], ib, isem).wait()

    def gather(ib, rows, gsem):
        pltpu.async_copy(table_hbm.at[ib.at[0]], rows, gsem)

    def gather_wait(ib, rows, gsem):
        pltpu.make_async_copy(table_hbm.at[ib.at[0]], rows, gsem).wait()

    def scatter(ib, rows, ssem):
        pltpu.async_copy(rows, acc.at[ib.at[1]], ssem, add=True)

    def scatter_wait(ib, rows, ssem):
        pltpu.make_async_copy(rows, acc.at[ib.at[1]], ssem).wait()

    idx_load(0, ib0, isem0)
    idx_load(1, ib1, isem1)
    idx_load(2, ib2, isem2)
    idx_load(3, ib3, isem3)
    idx_wait(0, ib0, isem0)
    gather(ib0, rows0, gsem0)
    idx_wait(1, ib1, isem1)
    gather(ib1, rows1, gsem1)

    def body(k, carry):
        i = 4 * k
        gather_wait(ib0, rows0, gsem0)
        scatter(ib0, rows0, ssem0)
        gather_wait(ib1, rows1, gsem1)
        scatter(ib1, rows1, ssem1)
        scatter_wait(ib0, rows0, ssem0)
        idx_load(i + 4, ib0, isem0)
        idx_wait(i + 2, ib2, isem2)
        gather(ib2, rows0, gsem0)
        scatter_wait(ib1, rows1, ssem1)
        idx_load(i + 5, ib1, isem1)
        idx_wait(i + 3, ib3, isem3)
        gather(ib3, rows1, gsem1)
        gather_wait(ib2, rows0, gsem0)
        scatter(ib2, rows0, ssem0)
        gather_wait(ib3, rows1, gsem1)
        scatter(ib3, rows1, ssem1)
        scatter_wait(ib2, rows0, ssem0)
        idx_load(i + 6, ib2, isem2)
        idx_wait(i + 4, ib0, isem0)
        gather(ib0, rows0, gsem0)
        scatter_wait(ib3, rows1, ssem1)
        idx_load(i + 7, ib3, isem3)
        idx_wait(i + 5, ib1, isem1)
        gather(ib1, rows1, gsem1)
        return carry

    lax.fori_loop(0, (NCH - 5) // 4, body, 0)
    gather_wait(ib0, rows0, gsem0)
    scatter(ib0, rows0, ssem0)
    gather_wait(ib1, rows1, gsem1)
    scatter(ib1, rows1, ssem1)
    scatter_wait(ib0, rows0, ssem0)
    idx_load(NCH - 1, ib0, isem0)
    idx_wait(NCH - 3, ib2, isem2)
    gather(ib2, rows0, gsem0)
    scatter_wait(ib1, rows1, ssem1)
    idx_wait(NCH - 2, ib3, isem3)
    gather(ib3, rows1, gsem1)
    gather_wait(ib2, rows0, gsem0)
    scatter(ib2, rows0, ssem0)
    gather_wait(ib3, rows1, gsem1)
    scatter(ib3, rows1, ssem1)
    scatter_wait(ib2, rows0, ssem0)
    idx_wait(NCH - 1, ib0, isem0)
    gather(ib0, rows0, gsem0)
    gather_wait(ib0, rows0, gsem0)
    scatter(ib0, rows0, ssem0)
    scatter_wait(ib3, rows1, ssem1)
    scatter_wait(ib0, rows0, ssem0)
    plsc.subcore_barrier()
    pltpu.sync_copy(acc.at[pl.ds(r0, RPT)], out_hbm.at[c, pl.ds(r0, RPT)])


_RB = 2000


def _rsqrt_deg(degp_ref):
    deg = degp_ref[0, :, 0:1] + degp_ref[1, :, 0:1] + 1.0
    return lax.rsqrt(deg)


def _tc1_body(x_ref, w1_ref, degp_ref, h1a_ref, h1b_ref):
    xh = jnp.dot(x_ref[...], w1_ref[...], preferred_element_type=jnp.float32)
    d = _rsqrt_deg(degp_ref)
    h = xh * d
    h1a_ref[...] = h[:, :H2]
    h1b_ref[...] = h[:, H2:]


def _tc2_body(a1a_ref, a1b_ref, h1a_ref, h1b_ref, degp_ref,
              w2a_ref, w2b_ref, b1_ref, h2_ref):
    d = _rsqrt_deg(degp_ref)
    ua = d * (a1a_ref[0] + a1a_ref[1] + h1a_ref[...]) + b1_ref[:, :H2]
    ub = d * (a1b_ref[0] + a1b_ref[1] + h1b_ref[...]) + b1_ref[:, H2:]
    ra = jnp.where(ua >= 0, ua, ua * SLOPE)
    rb = jnp.where(ub >= 0, ub, ub * SLOPE)
    xh2 = (jnp.dot(ra, w2a_ref[...], preferred_element_type=jnp.float32)
           + jnp.dot(rb, w2b_ref[...], preferred_element_type=jnp.float32))
    h2_ref[...] = xh2 * d


def _tc3_body(a2_ref, h2_ref, degp_ref, b2_ref, z_ref):
    d = _rsqrt_deg(degp_ref)
    z_ref[...] = d * (a2_ref[0] + a2_ref[1] + h2_ref[...]) + b2_ref[...]


def _row_spec(w):
    return pl.BlockSpec((_RB, w), lambda i: (i, 0))


def _part_spec(w):
    return pl.BlockSpec((NC, _RB, w), lambda i: (0, i, 0))


_DEG_SPEC = pl.BlockSpec((NC, _RB, DEGW), lambda i: (0, i, 0))
_GRID = (N // _RB,)

_tc1 = pl.pallas_call(
    _tc1_body,
    grid=_GRID,
    in_specs=[
        _row_spec(IN_CH),
        pl.BlockSpec((IN_CH, H1), lambda i: (0, 0)),
        _DEG_SPEC,
    ],
    out_specs=[_row_spec(H2), _row_spec(H2)],
    out_shape=[
        jax.ShapeDtypeStruct((N, H2), jnp.float32),
        jax.ShapeDtypeStruct((N, H2), jnp.float32),
    ],
)

_tc2 = pl.pallas_call(
    _tc2_body,
    grid=_GRID,
    in_specs=[
        _part_spec(H2),
        _part_spec(H2),
        _row_spec(H2),
        _row_spec(H2),
        _DEG_SPEC,
        pl.BlockSpec((H2, H2), lambda i: (0, 0)),
        pl.BlockSpec((H2, H2), lambda i: (0, 0)),
        pl.BlockSpec((1, H1), lambda i: (0, 0)),
    ],
    out_specs=_row_spec(H2),
    out_shape=jax.ShapeDtypeStruct((N, H2), jnp.float32),
)

_tc3 = pl.pallas_call(
    _tc3_body,
    grid=_GRID,
    in_specs=[
        _part_spec(H2),
        _row_spec(H2),
        _DEG_SPEC,
        pl.BlockSpec((1, H2), lambda i: (0, 0)),
    ],
    out_specs=_row_spec(H2),
    out_shape=jax.ShapeDtypeStruct((N, H2), jnp.float32),
)


def kernel(x, edge_index, W1, b1, W2, b2):
    src = edge_index[0].astype(jnp.int32)
    dst = edge_index[1].astype(jnp.int32)
    ones_rows = jnp.ones((DCH, DEGW), jnp.float32)
    zdeg = jnp.zeros((ACC_N, DEGW), jnp.float32)
    zacc = jnp.zeros((ACC_N, H2), jnp.float32)

    src3 = src.reshape(NW, NCH, CH)
    dst3 = dst.reshape(NW, NCH, CH)
    idx2 = jnp.stack([src3, dst3], axis=2)
    degp = _deg_kernel(dst3, ones_rows, zdeg)
    h1a, h1b = _tc1(x, W1, degp)
    agg1a = _spmm_kernel(h1a, idx2, zacc)
    agg1b = _spmm_kernel(h1b, idx2, zacc)
    h2 = _tc2(agg1a, agg1b, h1a, h1b, degp,
              W2[:H2], W2[H2:], b1.reshape(1, H1))
    agg2 = _spmm_kernel(h2, idx2, zacc)
    z = _tc3(agg2, h2, degp, b2.reshape(1, H2))
    return z


# speedup vs baseline: 4.2246x; 1.2531x over previous
import functools

import jax
import jax.numpy as jnp
from jax import lax
from jax.experimental import pallas as pl
from jax.experimental.pallas import tpu as pltpu
from jax.experimental.pallas import tpu_sc as plsc

N = 10000
E = 320000
IN_CH = 128
H1 = 256
H2 = 128
NC, NS = 2, 16
NW = NC * NS
CH = 80
EPT = E // NW
NCH = EPT // CH
ACC_N = 10240
RPT = ACC_N // NS
DCH = 80
DNCH = E // NW // DCH
DEGW = 128
SLOPE = (1.0 / 8 + 1.0 / 3) / 2.0

_MESH = plsc.VectorSubcoreMesh(
    core_axis_name="c", subcore_axis_name="s", num_cores=NC, num_subcores=NS
)


@functools.partial(
    pl.kernel,
    out_type=jax.ShapeDtypeStruct((NC, ACC_N, DEGW), jnp.float32),
    mesh=_MESH,
    scratch_types=[
        pltpu.VMEM((NCH, CH), jnp.int32),
        pltpu.VMEM((DCH, DEGW), jnp.float32),
        pltpu.VMEM_SHARED((ACC_N, DEGW), jnp.float32),
        pltpu.SemaphoreType.DMA,
        pltpu.SemaphoreType.DMA,
    ],
)
def _deg_kernel(dst3_hbm, ones_hbm, zero_hbm, out_hbm, didx_all, ones_v, acc,
                ssem0, ssem1):
    c = lax.axis_index("c")
    s = lax.axis_index("s")
    w = c * NS + s
    r0 = s * RPT
    pltpu.sync_copy(ones_hbm, ones_v)
    pltpu.sync_copy(dst3_hbm.at[w], didx_all)
    pltpu.sync_copy(zero_hbm.at[pl.ds(r0, RPT)], acc.at[pl.ds(r0, RPT)])
    plsc.subcore_barrier()

    pltpu.async_copy(ones_v, acc.at[didx_all.at[0]], ssem0, add=True)
    pltpu.async_copy(ones_v, acc.at[didx_all.at[1]], ssem1, add=True)

    def body(j, carry):
        pltpu.make_async_copy(ones_v, acc.at[didx_all.at[2 * j]], ssem0).wait()
        pltpu.async_copy(ones_v, acc.at[didx_all.at[2 * j + 2]], ssem0, add=True)
        pltpu.make_async_copy(ones_v, acc.at[didx_all.at[2 * j + 1]], ssem1).wait()
        pltpu.async_copy(ones_v, acc.at[didx_all.at[2 * j + 3]], ssem1, add=True)
        return carry

    lax.fori_loop(0, NCH // 2 - 1, body, 0)
    pltpu.make_async_copy(ones_v, acc.at[didx_all.at[NCH - 3]], ssem0).wait()
    pltpu.async_copy(ones_v, acc.at[didx_all.at[NCH - 1]], ssem0, add=True)
    pltpu.make_async_copy(ones_v, acc.at[didx_all.at[NCH - 2]], ssem1).wait()
    pltpu.make_async_copy(ones_v, acc.at[didx_all.at[NCH - 1]], ssem0).wait()
    plsc.subcore_barrier()
    pltpu.sync_copy(acc.at[pl.ds(r0, RPT)], out_hbm.at[c, pl.ds(r0, RPT)])


def _spmm_body(table_hbm, src3_hbm, dst3_hbm, zero_hbm, out_hbm,
               sidx, didx, rows, acc, gsems, ssems, sisems, disems):
    c = lax.axis_index("c")
    s = lax.axis_index("s")
    w = c * NS + s
    r0 = s * RPT
    pltpu.sync_copy(zero_hbm.at[pl.ds(r0, RPT)], acc.at[pl.ds(r0, RPT)])
    plsc.subcore_barrier()

    def sload(i, t):
        pltpu.async_copy(src3_hbm.at[w, i], sidx[t], sisems[t])

    def swait(i, t):
        pltpu.make_async_copy(src3_hbm.at[w, i], sidx[t], sisems[t]).wait()

    def dload(i, t):
        pltpu.async_copy(dst3_hbm.at[w, i], didx[t], disems[t])

    def dwait(i, t):
        pltpu.make_async_copy(dst3_hbm.at[w, i], didx[t], disems[t]).wait()

    def gather(t):
        pltpu.async_copy(table_hbm.at[sidx[t].at[0]], rows[t], gsems[t])

    def gather_wait(t):
        pltpu.make_async_copy(table_hbm.at[sidx[t].at[0]], rows[t], gsems[t]).wait()

    def scatter(t):
        pltpu.async_copy(rows[t], acc.at[didx[t].at[0]], ssems[t], add=True)

    def scatter_wait(t):
        pltpu.make_async_copy(rows[t], acc.at[didx[t].at[0]], ssems[t]).wait()

    for t in range(4):
        sload(t, t)
        dload(t, t)
    for t in range(4):
        swait(t, t)
        gather(t)

    def body(k, carry):
        i = 4 * k
        for t in range(4):
            gather_wait(t)
            sload(i + 4 + t, t)
            dwait(i + t, t)
            scatter(t)
        for t in range(4):
            scatter_wait(t)
            dload(i + 4 + t, t)
            swait(i + 4 + t, t)
            gather(t)
        return carry

    lax.fori_loop(0, (NCH - 5) // 4, body, 0)
    L = NCH - 5
    gather_wait(0)
    sload(NCH - 1, 0)
    dwait(L, 0)
    scatter(0)
    for t in range(1, 4):
        gather_wait(t)
        dwait(L + t, t)
        scatter(t)
    scatter_wait(0)
    dload(NCH - 1, 0)
    swait(NCH - 1, 0)
    gather(0)
    gather_wait(0)
    dwait(NCH - 1, 0)
    scatter(0)
    for t in range(1, 4):
        scatter_wait(t)
    scatter_wait(0)
    plsc.subcore_barrier()
    pltpu.sync_copy(acc.at[pl.ds(r0, RPT)], out_hbm.at[c, pl.ds(r0, RPT)])


@functools.partial(
    pl.kernel,
    out_type=jax.ShapeDtypeStruct((NC, ACC_N, H2), jnp.float32),
    mesh=_MESH,
    scratch_types=dict(
        sidx=[pltpu.VMEM((1, CH), jnp.int32) for _ in range(4)],
        didx=[pltpu.VMEM((1, CH), jnp.int32) for _ in range(4)],
        rows=[pltpu.VMEM((CH, H2), jnp.float32) for _ in range(4)],
        acc=pltpu.VMEM_SHARED((ACC_N, H2), jnp.float32),
        gsems=[pltpu.SemaphoreType.DMA for _ in range(4)],
        ssems=[pltpu.SemaphoreType.DMA for _ in range(4)],
        sisems=[pltpu.SemaphoreType.DMA for _ in range(4)],
        disems=[pltpu.SemaphoreType.DMA for _ in range(4)],
    ),
)
def _spmm_kernel(table_hbm, src3_hbm, dst3_hbm, zero_hbm, out_hbm, *,
                 sidx, didx, rows, acc, gsems, ssems, sisems, disems):
    _spmm_body(table_hbm, src3_hbm, dst3_hbm, zero_hbm, out_hbm,
               sidx, didx, rows, acc, gsems, ssems, sisems, disems)


_RB = 2000


def _rsqrt_deg(degp_ref):
    deg = degp_ref[0, :, 0:1] + degp_ref[1, :, 0:1] + 1.0
    return lax.rsqrt(deg)


def _tc1_body(x_ref, w1_ref, degp_ref, h1a_ref, h1b_ref):
    xh = jnp.dot(x_ref[...], w1_ref[...], preferred_element_type=jnp.float32)
    d = _rsqrt_deg(degp_ref)
    h = xh * d
    h1a_ref[...] = h[:, :H2]
    h1b_ref[...] = h[:, H2:]


def _tc2_body(a1a_ref, a1b_ref, h1a_ref, h1b_ref, degp_ref,
              w2a_ref, w2b_ref, b1_ref, h2_ref):
    d = _rsqrt_deg(degp_ref)
    ua = d * (a1a_ref[0] + a1a_ref[1] + h1a_ref[...]) + b1_ref[:, :H2]
    ub = d * (a1b_ref[0] + a1b_ref[1] + h1b_ref[...]) + b1_ref[:, H2:]
    ra = jnp.where(ua >= 0, ua, ua * SLOPE)
    rb = jnp.where(ub >= 0, ub, ub * SLOPE)
    xh2 = (jnp.dot(ra, w2a_ref[...], preferred_element_type=jnp.float32)
           + jnp.dot(rb, w2b_ref[...], preferred_element_type=jnp.float32))
    h2_ref[...] = xh2 * d


def _tc3_body(a2_ref, h2_ref, degp_ref, b2_ref, z_ref):
    d = _rsqrt_deg(degp_ref)
    z_ref[...] = d * (a2_ref[0] + a2_ref[1] + h2_ref[...]) + b2_ref[...]


def _row_spec(w):
    return pl.BlockSpec((_RB, w), lambda i: (i, 0))


def _part_spec(w):
    return pl.BlockSpec((NC, _RB, w), lambda i: (0, i, 0))


_DEG_SPEC = pl.BlockSpec((NC, _RB, DEGW), lambda i: (0, i, 0))
_GRID = (N // _RB,)

_tc1 = pl.pallas_call(
    _tc1_body,
    grid=_GRID,
    in_specs=[
        _row_spec(IN_CH),
        pl.BlockSpec((IN_CH, H1), lambda i: (0, 0)),
        _DEG_SPEC,
    ],
    out_specs=[_row_spec(H2), _row_spec(H2)],
    out_shape=[
        jax.ShapeDtypeStruct((N, H2), jnp.float32),
        jax.ShapeDtypeStruct((N, H2), jnp.float32),
    ],
)

_tc2 = pl.pallas_call(
    _tc2_body,
    grid=_GRID,
    in_specs=[
        _part_spec(H2),
        _part_spec(H2),
        _row_spec(H2),
        _row_spec(H2),
        _DEG_SPEC,
        pl.BlockSpec((H2, H2), lambda i: (0, 0)),
        pl.BlockSpec((H2, H2), lambda i: (0, 0)),
        pl.BlockSpec((1, H1), lambda i: (0, 0)),
    ],
    out_specs=_row_spec(H2),
    out_shape=jax.ShapeDtypeStruct((N, H2), jnp.float32),
)

_tc3 = pl.pallas_call(
    _tc3_body,
    grid=_GRID,
    in_specs=[
        _part_spec(H2),
        _row_spec(H2),
        _DEG_SPEC,
        pl.BlockSpec((1, H2), lambda i: (0, 0)),
    ],
    out_specs=_row_spec(H2),
    out_shape=jax.ShapeDtypeStruct((N, H2), jnp.float32),
)


def kernel(x, edge_index, W1, b1, W2, b2):
    src = edge_index[0].astype(jnp.int32)
    dst = edge_index[1].astype(jnp.int32)
    ones_rows = jnp.ones((DCH, DEGW), jnp.float32)
    zdeg = jnp.zeros((ACC_N, DEGW), jnp.float32)
    zacc = jnp.zeros((ACC_N, H2), jnp.float32)

    src3 = src.reshape(NW, NCH, 1, CH)
    dst3 = dst.reshape(NW, NCH, 1, CH)
    degp = _deg_kernel(dst3.reshape(NW, NCH, CH), ones_rows, zdeg)
    h1a, h1b = _tc1(x, W1, degp)
    agg1a = _spmm_kernel(h1a, src3, dst3, zacc)
    agg1b = _spmm_kernel(h1b, src3, dst3, zacc)
    h2 = _tc2(agg1a, agg1b, h1a, h1b, degp,
              W2[:H2], W2[H2:], b1.reshape(1, H1))
    agg2 = _spmm_kernel(h2, src3, dst3, zacc)
    z = _tc3(agg2, h2, degp, b2.reshape(1, H2))
    return z
